# baseline (speedup 1.0000x reference)
import functools

import jax
import jax.numpy as jnp
from jax import lax
from jax.experimental import pallas as pl
from jax.experimental.pallas import tpu as pltpu
from jax.experimental.pallas import tpu_sc as plsc

B = 8
NPG = 1024
K = 16
CIN = 256
COUT = 10
CPAD = 16
N = B * NPG
NEG = -3.0e38

RB = 16
NCH = NPG // RB
NW = 32
RPW = N // NW
NBLK = RPW // RB


def _mlp_body(x_ref, w1_ref, b1_ref, gamma_ref, beta_ref, w2_ref, bn_ref, emb_ref):
    x = x_ref[...]
    h = jnp.dot(x, w1_ref[...], preferred_element_type=jnp.float32) + b1_ref[...]
    mu = jnp.mean(h, axis=0, keepdims=True)
    var = jnp.mean((h - mu) ** 2, axis=0, keepdims=True)
    h = (h - mu) / jnp.sqrt(var + 1e-5) * gamma_ref[...] + beta_ref[...]
    h = jnp.maximum(h, 0.0)
    emb_ref[...] = jnp.dot(h, w2_ref[...], preferred_element_type=jnp.float32) + bn_ref[...]


def _d2_body(pos_ref, post_ref, out_ref):
    ri = lax.broadcasted_iota(jnp.int32, (NPG, NPG), 0)
    ci = lax.broadcasted_iota(jnp.int32, (NPG, NPG), 1)
    pos = pos_ref[0]
    post = post_ref[0]
    s_row = jnp.sum(pos * pos, axis=1, keepdims=True)
    s_col = jnp.sum(post * post, axis=0, keepdims=True)
    gram = jnp.dot(pos, post, preferred_element_type=jnp.float32)
    nd2 = jnp.minimum(2.0 * gram - s_row - s_col, 0.0)
    out_ref[...] = jnp.where(ri == ci, -1e12, nd2)


def _logits_body(t_ref, emb_ref, embt_ref, u_ref, out_ref):
    emb = emb_ref[0]
    embt = embt_ref[0]
    e_row = jnp.sum(emb * emb, axis=1, keepdims=True)
    e_col = jnp.sum(embt * embt, axis=0, keepdims=True)
    egram = jnp.dot(emb, embt, preferred_element_type=jnp.float32)
    ed2 = jnp.maximum(e_row + e_col - 2.0 * egram, 0.0)
    t = t_ref[0, 0]
    p = jnp.exp(-t * (ed2 + 1e-12))
    g = -jnp.log(-jnp.log(u_ref[0] + 1e-20) + 1e-20)
    out_ref[...] = jnp.log(p + 1e-20) + g


def _tournament(pairs):
    while len(pairs) > 1:
        nxt = []
        for i in range(0, len(pairs) - 1, 2):
            (va, ia), (vb, ib) = pairs[i], pairs[i + 1]
            upd = vb > va
            nxt.append((jnp.where(upd, vb, va), jnp.where(upd, ib, ia)))
        if len(pairs) % 2:
            nxt.append(pairs[-1])
        pairs = nxt
    return pairs[0]


NQ = 4
QC = NPG // NQ
NLG = 8
NQG = RPW // (NLG * RB)


def _sc_topk_body(with_vals, s_hbm, oi_hbm, ov_hbm, buf0, buf1, m_v, a_v,
                  thv, thi, oi_all, ov_all, sem0, sem1):
    i32 = jnp.int32
    f32 = jnp.float32
    cid = lax.axis_index("c")
    sid = lax.axis_index("s")
    wid = sid * 2 + cid
    base = wid * RPW
    g = base // NPG
    off = g * NPG
    lanes = lax.iota(i32, RB)
    bufs = (buf0, buf1)
    sems = (sem0, sem1)

    def src(s):
        qg, q = s // NQ, s % NQ
        return s_hbm.at[pl.ds(q * QC, QC), pl.ds(base + qg * 128, 128)]

    def phases(buf, q):
        def lg_body(lg, carry0):
            clane = lg * RB + lanes
            mrow = lg * RB

            def p1(c, carry1):
                col0 = c * RB
                pairs = []
                for tt in range(RB):
                    v = plsc.load_gather(
                        buf, [jnp.full((RB,), col0 + tt, i32), clane])
                    pairs.append((v, jnp.full((RB,), tt, i32)))
                m, t_win = _tournament(pairs)
                m_v[mrow + c, :] = m
                a_v[mrow + c, :] = t_win + col0
                return carry1

            lax.fori_loop(0, QC // RB, p1, 0)

            def p2(kk, carry2):
                pairs = [(m_v[mrow + c, :], jnp.full((RB,), c, i32))
                         for c in range(QC // RB)]
                m, cs = _tournament(pairs)
                astar = plsc.load_gather(a_v, [mrow + cs, lanes])
                trow = ((lg * NQ + q) * K + kk) * RB
                thv[pl.ds(trow, RB)] = m
                thi[pl.ds(trow, RB)] = astar + (q * QC + off)
                plsc.store_scatter(buf, [astar, clane],
                                   jnp.full((RB,), NEG, f32))
                colb = cs * RB
                pairs = []
                for tt in range(RB):
                    v = plsc.load_gather(buf, [colb + tt, clane])
                    pairs.append((v, jnp.full((RB,), tt, i32)))
                m2, t2 = _tournament(pairs)
                plsc.store_scatter(m_v, [mrow + cs, lanes], m2)
                plsc.store_scatter(a_v, [mrow + cs, lanes], t2 + colb)
                return carry2

            lax.fori_loop(0, K, p2, 0)
            return carry0

        lax.fori_loop(0, NLG, lg_body, 0)

    def merge(qg):
        def lg_body(lg, carry0):
            tb = [jnp.full((RB,), ((lg * NQ + q) * K) * RB, i32)
                  for q in range(NQ)]
            ptr = [jnp.zeros((RB,), i32) for _ in range(NQ)]
            obase = ((qg * NLG + lg) * K) * RB
            v0 = None
            for kk in range(K):
                rows = [tb[q] + ptr[q] * RB for q in range(NQ)]
                heads = [plsc.load_gather(thv, [rows[q] + lanes])
                         for q in range(NQ)]
                hv, hrow = _tournament(
                    [(heads[q], rows[q]) for q in range(NQ)])
                hi = plsc.load_gather(thi, [hrow + lanes])
                oi_all[pl.ds(obase + kk * RB, RB)] = hi
                if with_vals:
                    if kk == 0:
                        v0 = hv
                    ov_all[pl.ds(obase + kk * RB, RB)] = jnp.exp(hv - v0)
                ptr = [jnp.where(hrow == rows[q], ptr[q] + 1, ptr[q])
                       for q in range(NQ)]
            return carry0

        lax.fori_loop(0, NLG, lg_body, 0)

    pltpu.make_async_copy(src(0), buf0, sem0).start()
    for s in range(NQG * NQ):
        p = s % 2
        if s + 1 < NQG * NQ:
            pltpu.make_async_copy(src(s + 1), bufs[1 - p], sems[1 - p]).start()
        pltpu.make_async_copy(src(s), bufs[p], sems[p]).wait()
        phases(bufs[p], s % NQ)
        if s % NQ == NQ - 1:
            merge(s // NQ)

    wbase = wid * (NBLK * K * RB)
    pltpu.sync_copy(oi_all, oi_hbm.at[pl.ds(wbase, NBLK * K * RB)])
    if with_vals:
        pltpu.sync_copy(ov_all, ov_hbm.at[pl.ds(wbase, NBLK * K * RB)])


def _sc_topk(scores, with_vals):
    i32 = jnp.int32
    f32 = jnp.float32
    out_type = [jax.ShapeDtypeStruct((N * K,), i32)]
    scratch = [
        pltpu.VMEM((QC, 128), f32),
        pltpu.VMEM((QC, 128), f32),
        pltpu.VMEM((NLG * QC // RB, RB), f32),
        pltpu.VMEM((NLG * QC // RB, RB), i32),
        pltpu.VMEM((NLG * NQ * K * RB,), f32),
        pltpu.VMEM((NLG * NQ * K * RB,), i32),
        pltpu.VMEM((NBLK * K * RB,), i32),
        pltpu.VMEM((NBLK * K * RB,), f32),
        pltpu.SemaphoreType.DMA,
        pltpu.SemaphoreType.DMA,
    ]
    if with_vals:
        out_type.append(jax.ShapeDtypeStruct((N * K,), f32))
        body = functools.partial(_sc_topk_body, True)
    else:
        def body(s_hbm, oi_hbm, buf0, buf1, m_v, a_v, thv, thi, oi_all,
                 ov_all, s0, s1):
            return _sc_topk_body(False, s_hbm, oi_hbm, None, buf0, buf1, m_v,
                                 a_v, thv, thi, oi_all, ov_all, s0, s1)
    mesh = plsc.VectorSubcoreMesh(core_axis_name="c", subcore_axis_name="s",
                                  num_cores=2, num_subcores=16)
    fn = pl.kernel(body, out_type=out_type, mesh=mesh, scratch_types=scratch,
                   compiler_params=pltpu.CompilerParams(
                       use_tc_tiling_on_sc=True, needs_layout_passes=False),
                   cost_estimate=pl.CostEstimate(
                       flops=N * NPG * 2, bytes_accessed=N * NPG * 4,
                       transcendentals=N * K if with_vals else 0))
    return fn(scores)


def _unslot(x):
    return jnp.swapaxes(x.reshape(N // RB, K, RB), 1, 2).reshape(N, K)


def kernel(x, pos, batch, W1, b1, gamma, beta, W2, b2, t):
    f32 = jnp.float32
    nz = jax.random.uniform(jax.random.key(1), (N, COUT), dtype=f32) * 0.001
    u = jax.random.uniform(jax.random.key(2), (B, NPG, NPG), dtype=f32)
    w2p = jnp.pad(W2, ((0, 0), (0, CPAD - COUT)))
    bn = jnp.pad(b2[None, :] + nz, ((0, 0), (0, CPAD - COUT)))

    posp = jnp.pad(pos, ((0, 0), (0, 5))).reshape(B, NPG, 8)
    post = jnp.swapaxes(posp, 1, 2)
    g3 = lambda i: (i, 0, 0)

    sknn = pl.pallas_call(
        _d2_body,
        grid=(B,),
        in_specs=[
            pl.BlockSpec((1, NPG, 8), g3),
            pl.BlockSpec((1, 8, NPG), g3),
        ],
        out_specs=pl.BlockSpec((NPG, NPG), lambda i: (0, i)),
        out_shape=jax.ShapeDtypeStruct((NPG, N), f32),
    )(posp, post)
    knn_idx, = _sc_topk(sknn, with_vals=False)
    knn_idx = _unslot(knn_idx)

    emb = pl.pallas_call(
        _mlp_body,
        out_shape=jax.ShapeDtypeStruct((N, CPAD), f32),
    )(x, W1, b1[None, :], gamma[None, :], beta[None, :], w2p, bn)
    embr = emb.reshape(B, NPG, CPAD)
    embt = jnp.swapaxes(embr, 1, 2)

    sgum = pl.pallas_call(
        _logits_body,
        grid=(B,),
        in_specs=[
            pl.BlockSpec((1, 1), lambda i: (0, 0)),
            pl.BlockSpec((1, NPG, CPAD), g3),
            pl.BlockSpec((1, CPAD, NPG), g3),
            pl.BlockSpec((1, NPG, NPG), g3),
        ],
        out_specs=pl.BlockSpec((NPG, NPG), lambda i: (0, i)),
        out_shape=jax.ShapeDtypeStruct((NPG, N), f32),
    )(t.reshape(1, 1), embr, embt, u)
    gum_idx, gum_val = _sc_topk(sgum, with_vals=True)
    gum_idx = _unslot(gum_idx)
    gum_val = _unslot(gum_val)

    rows = jnp.repeat(jnp.arange(N, dtype=jnp.int32), K)
    knn_edge = jnp.stack([knn_idx.reshape(-1), rows], axis=0)
    soft_index_i = jnp.stack([gum_idx.reshape(-1), rows], axis=0)
    soft_index_v = jnp.stack([gum_val.reshape(-1), rows.astype(f32)], axis=0)
    edge_index = jnp.concatenate([soft_index_i, knn_edge], axis=1)
    return edge_index, soft_index_i, soft_index_v


# speedup vs baseline: 1.0235x; 1.0235x over previous
import functools

import jax
import jax.numpy as jnp
from jax import lax
from jax.experimental import pallas as pl
from jax.experimental.pallas import tpu as pltpu
from jax.experimental.pallas import tpu_sc as plsc

B = 8
NPG = 1024
K = 16
CIN = 256
COUT = 10
CPAD = 16
N = B * NPG
NEG = -3.0e38

RB = 16
NCH = NPG // RB
NW = 32
RPW = N // NW
NBLK = RPW // RB


def _mlp_body(x_ref, w1_ref, b1_ref, gamma_ref, beta_ref, w2_ref, bn_ref, emb_ref):
    x = x_ref[...]
    h = jnp.dot(x, w1_ref[...], preferred_element_type=jnp.float32) + b1_ref[...]
    mu = jnp.mean(h, axis=0, keepdims=True)
    var = jnp.mean((h - mu) ** 2, axis=0, keepdims=True)
    h = (h - mu) / jnp.sqrt(var + 1e-5) * gamma_ref[...] + beta_ref[...]
    h = jnp.maximum(h, 0.0)
    emb_ref[...] = jnp.dot(h, w2_ref[...], preferred_element_type=jnp.float32) + bn_ref[...]


def _d2_body(pos_ref, post_ref, out_ref):
    ri = lax.broadcasted_iota(jnp.int32, (NPG, NPG), 0)
    ci = lax.broadcasted_iota(jnp.int32, (NPG, NPG), 1)
    pos = pos_ref[0]
    post = post_ref[0]
    s_row = jnp.sum(pos * pos, axis=1, keepdims=True)
    s_col = jnp.sum(post * post, axis=0, keepdims=True)
    gram = jnp.dot(pos, post, preferred_element_type=jnp.float32)
    nd2 = jnp.minimum(2.0 * gram - s_row - s_col, 0.0)
    out_ref[0] = jnp.where(ri == ci, -1e12, nd2)


def _logits_body(t_ref, emb_ref, embt_ref, u_ref, out_ref):
    emb = emb_ref[0]
    embt = embt_ref[0]
    e_row = jnp.sum(emb * emb, axis=1, keepdims=True)
    e_col = jnp.sum(embt * embt, axis=0, keepdims=True)
    egram = jnp.dot(emb, embt, preferred_element_type=jnp.float32)
    ed2 = jnp.maximum(e_row + e_col - 2.0 * egram, 0.0)
    t = t_ref[0, 0]
    p = jnp.exp(-t * (ed2 + 1e-12))
    g = -jnp.log(-jnp.log(u_ref[0] + 1e-20) + 1e-20)
    out_ref[0] = jnp.log(p + 1e-20) + g


def _tournament(pairs):
    while len(pairs) > 1:
        nxt = []
        for i in range(0, len(pairs) - 1, 2):
            (va, ia), (vb, ib) = pairs[i], pairs[i + 1]
            upd = vb > va
            nxt.append((jnp.where(upd, vb, va), jnp.where(upd, ib, ia)))
        if len(pairs) % 2:
            nxt.append(pairs[-1])
        pairs = nxt
    return pairs[0]


def _sc_topk_body(with_vals, s_hbm, oi_hbm, ov_hbm, buf0, buf1, m_v, a_v,
                  oi_all, ov_all, sem0, sem1):
    i32 = jnp.int32
    f32 = jnp.float32
    cid = lax.axis_index("c")
    sid = lax.axis_index("s")
    wid = sid * 2 + cid
    base = wid * RPW
    g = base // NPG
    off = g * NPG
    rl_base = base - off
    lanes = lax.iota(i32, RB)
    bufs = (buf0, buf1)
    sems = (sem0, sem1)

    def src(bi):
        return s_hbm.at[pl.ds(g * NPG, NPG), pl.ds(rl_base + bi * RB, RB)]

    def process(buf, bi):
        def p1(c, carry1):
            col0 = c * RB
            pairs = []
            for tt in range(RB):
                v = plsc.load_gather(buf, [jnp.full((RB,), col0 + tt, i32),
                                           lanes])
                pairs.append((v, jnp.full((RB,), tt, i32)))
            m, t_win = _tournament(pairs)
            m_v[c, :] = m
            a_v[c, :] = t_win + col0
            return carry1

        lax.fori_loop(0, NCH, p1, 0)

        def p2(kk, carry2):
            pairs = [(m_v[c, :], jnp.full((RB,), c, i32)) for c in range(NCH)]
            m, cs = _tournament(pairs)
            astar = plsc.load_gather(a_v, [cs, lanes])
            obase = (bi * K + kk) * RB
            oi_all[pl.ds(obase, RB)] = astar + off
            if with_vals:
                ov_all[pl.ds(obase, RB)] = m
            plsc.store_scatter(buf, [astar, lanes], jnp.full((RB,), NEG, f32))
            colb = cs * RB
            pairs = []
            for tt in range(RB):
                v = plsc.load_gather(buf, [colb + tt, lanes])
                pairs.append((v, jnp.full((RB,), tt, i32)))
            m2, t2 = _tournament(pairs)
            plsc.store_scatter(m_v, [cs, lanes], m2)
            plsc.store_scatter(a_v, [cs, lanes], t2 + colb)
            return carry2

        lax.fori_loop(0, K, p2, 0)

        if with_vals:
            v0 = ov_all[pl.ds(bi * K * RB, RB)]

            def sm(kk, carry3):
                obase = (bi * K + kk) * RB
                v = ov_all[pl.ds(obase, RB)]
                ov_all[pl.ds(obase, RB)] = jnp.exp(v - v0)
                return carry3

            lax.fori_loop(0, K, sm, 0)

    pltpu.make_async_copy(src(0), buf0, sem0).start()

    def pair(sb, carry):
        for p in (0, 1):
            bi = sb * 2 + p
            bj = bi + 1

            @pl.when(bj < NBLK)
            def _prefetch():
                pltpu.make_async_copy(src(bj), bufs[1 - p], sems[1 - p]).start()

            pltpu.make_async_copy(src(bi), bufs[p], sems[p]).wait()
            process(bufs[p], bi)
        return carry

    lax.fori_loop(0, NBLK // 2, pair, 0)

    wbase = wid * (NBLK * K * RB)
    pltpu.sync_copy(oi_all, oi_hbm.at[pl.ds(wbase, NBLK * K * RB)])
    if with_vals:
        pltpu.sync_copy(ov_all, ov_hbm.at[pl.ds(wbase, NBLK * K * RB)])


def _sc_topk(scores, with_vals):
    i32 = jnp.int32
    f32 = jnp.float32
    out_type = [jax.ShapeDtypeStruct((N * K,), i32)]
    scratch = [
        pltpu.VMEM((NPG, RB), f32),
        pltpu.VMEM((NPG, RB), f32),
        pltpu.VMEM((NCH, RB), f32),
        pltpu.VMEM((NCH, RB), i32),
        pltpu.VMEM((NBLK * K * RB,), i32),
        pltpu.VMEM((NBLK * K * RB,), f32),
        pltpu.SemaphoreType.DMA,
        pltpu.SemaphoreType.DMA,
    ]
    if with_vals:
        out_type.append(jax.ShapeDtypeStruct((N * K,), f32))
        body = functools.partial(_sc_topk_body, True)
    else:
        def body(s_hbm, oi_hbm, buf0, buf1, m_v, a_v, oi_all, ov_all, s0, s1):
            return _sc_topk_body(False, s_hbm, oi_hbm, None, buf0, buf1, m_v,
                                 a_v, oi_all, ov_all, s0, s1)
    mesh = plsc.VectorSubcoreMesh(core_axis_name="c", subcore_axis_name="s",
                                  num_cores=2, num_subcores=16)
    fn = pl.kernel(body, out_type=out_type, mesh=mesh, scratch_types=scratch,
                   compiler_params=pltpu.CompilerParams(
                       use_tc_tiling_on_sc=False, needs_layout_passes=False),
                   cost_estimate=pl.CostEstimate(
                       flops=N * NPG * 2, bytes_accessed=N * NPG * 4,
                       transcendentals=N * K if with_vals else 0))
    return fn(scores)


def _unslot(x):
    return jnp.swapaxes(x.reshape(N // RB, K, RB), 1, 2).reshape(N, K)


def kernel(x, pos, batch, W1, b1, gamma, beta, W2, b2, t):
    f32 = jnp.float32
    nz = jax.random.uniform(jax.random.key(1), (N, COUT), dtype=f32) * 0.001
    u = jax.random.uniform(jax.random.key(2), (B, NPG, NPG), dtype=f32)
    w2p = jnp.pad(W2, ((0, 0), (0, CPAD - COUT)))
    bn = jnp.pad(b2[None, :] + nz, ((0, 0), (0, CPAD - COUT)))

    posp = jnp.pad(pos, ((0, 0), (0, 5))).reshape(B, NPG, 8)
    post = jnp.swapaxes(posp, 1, 2)
    g3 = lambda i: (i, 0, 0)

    sknn = pl.pallas_call(
        _d2_body,
        grid=(B,),
        in_specs=[
            pl.BlockSpec((1, NPG, 8), g3),
            pl.BlockSpec((1, 8, NPG), g3),
        ],
        out_specs=pl.BlockSpec((1, NPG, NPG), g3),
        out_shape=jax.ShapeDtypeStruct((B, NPG, NPG), f32),
    )(posp, post)
    knn_idx, = _sc_topk(sknn.reshape(N, NPG), with_vals=False)
    knn_idx = _unslot(knn_idx)

    emb = pl.pallas_call(
        _mlp_body,
        out_shape=jax.ShapeDtypeStruct((N, CPAD), f32),
    )(x, W1, b1[None, :], gamma[None, :], beta[None, :], w2p, bn)
    embr = emb.reshape(B, NPG, CPAD)
    embt = jnp.swapaxes(embr, 1, 2)

    sgum = pl.pallas_call(
        _logits_body,
        grid=(B,),
        in_specs=[
            pl.BlockSpec((1, 1), lambda i: (0, 0)),
            pl.BlockSpec((1, NPG, CPAD), g3),
            pl.BlockSpec((1, CPAD, NPG), g3),
            pl.BlockSpec((1, NPG, NPG), g3),
        ],
        out_specs=pl.BlockSpec((1, NPG, NPG), g3),
        out_shape=jax.ShapeDtypeStruct((B, NPG, NPG), f32),
    )(t.reshape(1, 1), embr, embt, u)
    gum_idx, gum_val = _sc_topk(sgum.reshape(N, NPG), with_vals=True)
    gum_idx = _unslot(gum_idx)
    gum_val = _unslot(gum_val)

    rows = jnp.repeat(jnp.arange(N, dtype=jnp.int32), K)
    knn_edge = jnp.stack([knn_idx.reshape(-1), rows], axis=0)
    soft_index_i = jnp.stack([gum_idx.reshape(-1), rows], axis=0)
    soft_index_v = jnp.stack([gum_val.reshape(-1), rows.astype(f32)], axis=0)
    edge_index = jnp.concatenate([soft_index_i, knn_edge], axis=1)
    return edge_index, soft_index_i, soft_index_v
